# Initial kernel scaffold; baseline (speedup 1.0000x reference)
import jax
import jax.numpy as jnp
from jax.experimental import pallas as pl


def kernel(raiser_pos, caller_pos, checked_to, hand, hands_strength, flops_strength, hand_pot, flop_pot, cards_ord, emb_raiser, emb_hand, emb_hs, emb_fs, emb_hp, emb_ord, W1, b1, W2, b2):
    raise NotImplementedError("write your pallas kernel here")


# speedup vs baseline: 6.2231x; 6.2231x over previous
import functools

import jax
import jax.numpy as jnp
from jax import lax
from jax.experimental import pallas as pl
from jax.experimental.pallas import tpu as pltpu
from jax.experimental.pallas import tpu_sc as plsc

B = 16384
NC, NS, L = 2, 16, 16
NW = NC * NS
RPW = B // NW
CHUNKS = RPW // L
ND = 9
NT = 896
OFF_RC, OFF_2, OFF_3 = 196, 232, 488


def _prep_body(w1aug_ref, eaug_ref, ft_ref):
    ft_ref[...] = jnp.dot(w1aug_ref[...], eaug_ref[...],
                          preferred_element_type=jnp.float32)


_prep = pl.pallas_call(
    _prep_body,
    out_shape=jax.ShapeDtypeStruct((ND, NT), jnp.float32),
)


_mesh = plsc.VectorSubcoreMesh(core_axis_name="c", subcore_axis_name="s",
                               num_cores=NC, num_subcores=NS)


@functools.partial(
    pl.kernel,
    out_type=jax.ShapeDtypeStruct((B * 3,), jnp.float32),
    mesh=_mesh,
    compiler_params=pltpu.CompilerParams(needs_layout_passes=False),
    scratch_types=[
        pltpu.VMEM((RPW,), jnp.int32),
        pltpu.VMEM((RPW,), jnp.int32),
        pltpu.VMEM((RPW,), jnp.int32),
        pltpu.VMEM((RPW,), jnp.int32),
        pltpu.VMEM((RPW,), jnp.int32),
        pltpu.VMEM((RPW,), jnp.int32),
        pltpu.VMEM((RPW,), jnp.int32),
        pltpu.VMEM((RPW,), jnp.int32),
        pltpu.VMEM((RPW,), jnp.float32),
        pltpu.VMEM((ND * NT,), jnp.float32),
        pltpu.VMEM((40, L), jnp.float32),
        pltpu.VMEM((RPW * 3,), jnp.float32),
        pltpu.SemaphoreType.DMA,
    ],
)
def _sc_main(rp_h, cp_h, hand_h, hs_h, fs_h, hp_h, fp_h, co_h, ct_h,
             ft_h, consts_h, out_h,
             rp_b, cp_b, hand_b, hs_b, fs_b, hp_b, fp_b, co_b, ct_b,
             ft_v, consts_v, out_v, sem):
    wid = lax.axis_index("s") * NC + lax.axis_index("c")
    base = wid * RPW

    copies = [
        pltpu.async_copy(src.at[pl.ds(base, RPW)], dst, sem)
        for src, dst in ((rp_h, rp_b), (cp_h, cp_b), (hand_h, hand_b),
                         (hs_h, hs_b), (fs_h, fs_b), (hp_h, hp_b),
                         (fp_h, fp_b), (co_h, co_b), (ct_h, ct_b))
    ]
    copies.append(pltpu.async_copy(ft_h, ft_v, sem))
    copies.append(pltpu.async_copy(consts_h, consts_v, sem))
    for c in copies:
        c.wait()

    w0 = [consts_v[d] for d in range(ND)]
    w2 = [consts_v[ND + i] for i in range(27)]
    b2 = [consts_v[ND + 27 + e] for e in range(3)]

    oidx0 = lax.iota(jnp.int32, L) * 3

    def chunk(c, oidx):
        s = c * L
        rp_v = rp_b[pl.ds(s, L)]
        cp_v = cp_b[pl.ds(s, L)]
        hand_v = hand_b[pl.ds(s, L)]
        hs_v = hs_b[pl.ds(s, L)]
        fs_v = fs_b[pl.ds(s, L)]
        hp_v = hp_b[pl.ds(s, L)]
        fp_v = fp_b[pl.ds(s, L)]
        co_v = co_b[pl.ds(s, L)]
        ct_v = ct_b[pl.ds(s, L)]

        i_h = hand_v
        i_rc = rp_v * 6 + cp_v + OFF_RC
        i_2 = hp_v * 16 + fp_v + OFF_2
        i_3 = hs_v * 40 + fs_v * 10 + (co_v + OFF_3)

        accs = [ct_v * w0[d] for d in range(ND)]
        for idx0 in (i_h, i_rc, i_2, i_3):
            idx = idx0
            for d in range(ND):
                g = plsc.load_gather(ft_v, [idx])
                accs[d] = accs[d] + g
                if d < ND - 1:
                    idx = idx + NT
        accs = [jnp.maximum(a, 0.0) for a in accs]
        for e in range(3):
            o = b2[e]
            for d in range(ND):
                o = o + accs[d] * w2[e * ND + d]
            plsc.store_scatter(out_v, [oidx + e], o)
        return oidx + 3 * L

    lax.fori_loop(0, CHUNKS, chunk, oidx0)
    pltpu.sync_copy(out_v, out_h.at[pl.ds(base * 3, RPW * 3)])


def kernel(raiser_pos, caller_pos, checked_to, hand, hands_strength,
           flops_strength, hand_pot, flop_pot, cards_ord,
           emb_raiser, emb_hand, emb_hs, emb_fs, emb_hp, emb_ord,
           W1, b1, W2, b2):
    f32 = jnp.float32
    i32 = jnp.int32

    r0 = emb_raiser[:, 0]
    rep_r = jnp.broadcast_to(r0[:, None], (6, 6)).reshape(1, 36)
    til_c = jnp.broadcast_to(r0[None, :], (6, 6)).reshape(1, 36)
    handT = emb_hand.T
    hsT, fsT, hpT, ordT = emb_hs.T, emb_fs.T, emb_hp.T, emb_ord.T
    hs_e = jnp.broadcast_to(hsT[:, :, None], (2, 10, 40)).reshape(2, 400)
    fs_e = jnp.broadcast_to(
        jnp.broadcast_to(fsT[:, :, None], (2, 4, 10)).reshape(2, 40)[:, None, :],
        (2, 10, 40)).reshape(2, 400)
    co_e = jnp.broadcast_to(ordT[:, None, :], (2, 40, 10)).reshape(2, 400)
    hp_e = jnp.broadcast_to(hpT[:, :, None], (2, 16, 16)).reshape(2, 256)
    fp_e = jnp.broadcast_to(hpT[:, None, :], (2, 16, 16)).reshape(2, 256)

    def z(r, c):
        return jnp.zeros((r, c), f32)

    blk_h = jnp.concatenate([z(2, 196), handT, z(10, 196),
                             jnp.ones((1, 196), f32)], axis=0)
    blk_rc = jnp.concatenate([rep_r, til_c, z(14, 36)], axis=0)
    blk_2 = jnp.concatenate([z(9, 256), hp_e, fp_e, z(3, 256)], axis=0)
    blk_3 = jnp.concatenate([z(5, 400), hs_e, fs_e, z(4, 400), co_e,
                             z(1, 400)], axis=0)
    eaug = jnp.concatenate([blk_h, blk_rc, blk_2, blk_3, z(16, 8)], axis=1)

    w1t = W1.T.astype(f32)
    w1aug = jnp.concatenate([w1t[:, 1:], b1.astype(f32)[:, None]], axis=1)

    ft = _prep(w1aug, eaug).reshape(ND * NT)

    cvec = jnp.concatenate([W1[0, :].astype(f32), W2.T.astype(f32).reshape(27),
                            b2.astype(f32), jnp.zeros((1,), f32)])
    consts = jnp.broadcast_to(cvec[:, None], (40, L))

    out_flat = _sc_main(raiser_pos.astype(i32), caller_pos.astype(i32),
                        hand.astype(i32), hands_strength.astype(i32),
                        flops_strength.astype(i32), hand_pot.astype(i32),
                        flop_pot.astype(i32), cards_ord.astype(i32),
                        checked_to.astype(f32), ft, consts)
    return out_flat.reshape(B, 3)


# speedup vs baseline: 6.9152x; 1.1112x over previous
import functools

import numpy as np

import jax
import jax.numpy as jnp
from jax import lax
from jax.experimental import pallas as pl
from jax.experimental.pallas import tpu as pltpu
from jax.experimental.pallas import tpu_sc as plsc

B = 16384
NC, NS, L = 2, 16, 16
NW = NC * NS
RPW = B // NW
CHUNKS = RPW // L
ND = 9
NT = 896
OFF_RC, OFF_2, OFF_3 = 196, 232, 488
BLK = 2048
NB = B // BLK
NDP = 16

_R6 = np.repeat(np.eye(6, dtype=np.float32), 6, axis=1)
_T6 = np.tile(np.eye(6, dtype=np.float32), (1, 6))
_R16 = np.repeat(np.eye(16, dtype=np.float32), 16, axis=1)
_T16 = np.tile(np.eye(16, dtype=np.float32), (1, 16))
_R40 = np.repeat(np.eye(10, dtype=np.float32), 40, axis=1)
_M40 = np.tile(np.repeat(np.eye(4, dtype=np.float32), 10, axis=1), (1, 10))
_T40 = np.tile(np.eye(10, dtype=np.float32), (1, 40))


def _dgc(a, b):
    return lax.dot_general(a, b, (((0,), (1,)), ((), ())),
                           preferred_element_type=jnp.float32)


def _mm(a, b):
    return lax.dot_general(a, b, (((1,), (0,)), ((), ())),
                           preferred_element_type=jnp.float32)


def _prep_body(er_ref, eh_ref, ehs_ref, efs_ref, ehp_ref, eord_ref, w1_ref,
               r6_ref, t6_ref, r16_ref, t16_ref, r40_ref, m40_ref, t40_ref,
               ft_ref):
    w1 = w1_ref[...]
    blk_h = _dgc(w1[3:6], eh_ref[...])
    g_r = _dgc(w1[1:2], er_ref[...])
    g_c = _dgc(w1[2:3], er_ref[...])
    blk_rc = _mm(g_r, r6_ref[...]) + _mm(g_c, t6_ref[...])
    g_hp = _dgc(w1[10:12], ehp_ref[...])
    g_fp = _dgc(w1[12:14], ehp_ref[...])
    blk_2 = _mm(g_hp, r16_ref[...]) + _mm(g_fp, t16_ref[...])
    g_hs = _dgc(w1[6:8], ehs_ref[...])
    g_fs = _dgc(w1[8:10], efs_ref[...])
    g_co = _dgc(w1[14:16], eord_ref[...])
    blk_3 = (_mm(g_hs, r40_ref[...]) + _mm(g_fs, m40_ref[...])
             + _mm(g_co, t40_ref[...]))
    ft_ref[...] = jnp.concatenate(
        [blk_h, blk_rc, blk_2, blk_3, jnp.zeros((ND, 8), jnp.float32)],
        axis=1)


_prep = pl.pallas_call(
    _prep_body,
    out_shape=jax.ShapeDtypeStruct((ND, NT), jnp.float32),
)


_mesh = plsc.VectorSubcoreMesh(core_axis_name="c", subcore_axis_name="s",
                               num_cores=NC, num_subcores=NS)


@functools.partial(
    pl.kernel,
    out_type=jax.ShapeDtypeStruct((NB * NDP * BLK,), jnp.float32),
    mesh=_mesh,
    compiler_params=pltpu.CompilerParams(needs_layout_passes=False),
    scratch_types=[
        pltpu.VMEM((RPW,), jnp.int32),
        pltpu.VMEM((RPW,), jnp.int32),
        pltpu.VMEM((RPW,), jnp.int32),
        pltpu.VMEM((RPW,), jnp.int32),
        pltpu.VMEM((RPW,), jnp.int32),
        pltpu.VMEM((RPW,), jnp.int32),
        pltpu.VMEM((RPW,), jnp.int32),
        pltpu.VMEM((RPW,), jnp.int32),
        pltpu.VMEM((ND * NT,), jnp.float32),
        pltpu.VMEM((ND * RPW,), jnp.float32),
        pltpu.SemaphoreType.DMA,
    ],
)
def _sc_main(rp_h, cp_h, hand_h, hs_h, fs_h, hp_h, fp_h, co_h, ft_h, h1_h,
             rp_b, cp_b, hand_b, hs_b, fs_b, hp_b, fp_b, co_b,
             ft_v, out_v, sem):
    wid = lax.axis_index("s") * NC + lax.axis_index("c")
    base = wid * RPW

    copies = [
        pltpu.async_copy(src.at[pl.ds(base, RPW)], dst, sem)
        for src, dst in ((rp_h, rp_b), (cp_h, cp_b), (hand_h, hand_b),
                         (hs_h, hs_b), (fs_h, fs_b), (hp_h, hp_b),
                         (fp_h, fp_b), (co_h, co_b))
    ]
    copies.append(pltpu.async_copy(ft_h, ft_v, sem))
    for c in copies:
        c.wait()

    def chunk(c, carry):
        s = c * L
        rp_v = rp_b[pl.ds(s, L)]
        cp_v = cp_b[pl.ds(s, L)]
        hand_v = hand_b[pl.ds(s, L)]
        hs_v = hs_b[pl.ds(s, L)]
        fs_v = fs_b[pl.ds(s, L)]
        hp_v = hp_b[pl.ds(s, L)]
        fp_v = fp_b[pl.ds(s, L)]
        co_v = co_b[pl.ds(s, L)]

        i_h = hand_v
        i_rc = rp_v * 6 + cp_v + OFF_RC
        i_2 = hp_v * 16 + fp_v + OFF_2
        i_3 = hs_v * 40 + fs_v * 10 + (co_v + OFF_3)

        accs = [None] * ND
        for idx0 in (i_h, i_rc, i_2, i_3):
            idx = idx0
            for d in range(ND):
                g = plsc.load_gather(ft_v, [idx])
                accs[d] = g if accs[d] is None else accs[d] + g
                if d < ND - 1:
                    idx = idx + NT
        for d in range(ND):
            out_v[pl.ds(d * RPW + s, L)] = accs[d]
        return carry

    lax.fori_loop(0, CHUNKS, chunk, 0)

    j = wid // 4
    col = (wid % 4) * RPW
    out_copies = [
        pltpu.async_copy(out_v.at[pl.ds(d * RPW, RPW)],
                         h1_h.at[pl.ds((j * NDP + d) * BLK + col, RPW)], sem)
        for d in range(ND)
    ]
    for c in out_copies:
        c.wait()


def _post_body(h1_ref, ct_ref, w1_ref, b1_ref, w2_ref, b2_ref, out_ref):
    a = jnp.concatenate([w1_ref[0:1, :], b1_ref[...]], axis=0)
    bm = jnp.concatenate([ct_ref[...], jnp.ones((1, BLK), jnp.float32)],
                         axis=0)
    h1 = h1_ref[0:ND, :] + lax.dot_general(
        a, bm, (((0,), (0,)), ((), ())), preferred_element_type=jnp.float32)
    r = jnp.maximum(h1, 0.0)
    out = lax.dot_general(r, w2_ref[...], (((0,), (0,)), ((), ())),
                          preferred_element_type=jnp.float32)
    out_ref[...] = out + b2_ref[...]


_post = pl.pallas_call(
    _post_body,
    grid=(NB,),
    in_specs=[
        pl.BlockSpec((NDP, BLK), lambda j: (j, 0)),
        pl.BlockSpec((1, BLK), lambda j: (0, j)),
        pl.BlockSpec((16, ND), lambda j: (0, 0)),
        pl.BlockSpec((1, ND), lambda j: (0, 0)),
        pl.BlockSpec((ND, 3), lambda j: (0, 0)),
        pl.BlockSpec((1, 3), lambda j: (0, 0)),
    ],
    out_specs=pl.BlockSpec((BLK, 3), lambda j: (j, 0)),
    out_shape=jax.ShapeDtypeStruct((B, 3), jnp.float32),
)


def kernel(raiser_pos, caller_pos, checked_to, hand, hands_strength,
           flops_strength, hand_pot, flop_pot, cards_ord,
           emb_raiser, emb_hand, emb_hs, emb_fs, emb_hp, emb_ord,
           W1, b1, W2, b2):
    f32 = jnp.float32
    i32 = jnp.int32

    ft = _prep(emb_raiser.astype(f32), emb_hand.astype(f32),
               emb_hs.astype(f32), emb_fs.astype(f32), emb_hp.astype(f32),
               emb_ord.astype(f32), W1.astype(f32),
               jnp.asarray(_R6), jnp.asarray(_T6), jnp.asarray(_R16),
               jnp.asarray(_T16), jnp.asarray(_R40), jnp.asarray(_M40),
               jnp.asarray(_T40))

    h1p = _sc_main(raiser_pos.astype(i32), caller_pos.astype(i32),
                   hand.astype(i32), hands_strength.astype(i32),
                   flops_strength.astype(i32), hand_pot.astype(i32),
                   flop_pot.astype(i32), cards_ord.astype(i32),
                   ft.reshape(ND * NT))

    return _post(h1p.reshape(NB * NDP, BLK),
                 checked_to.astype(f32).reshape(1, B), W1.astype(f32),
                 b1.astype(f32).reshape(1, ND), W2.astype(f32),
                 b2.astype(f32).reshape(1, 3))


# speedup vs baseline: 10.4619x; 1.5129x over previous
import functools

import jax
import jax.numpy as jnp
from jax import lax
from jax.experimental import pallas as pl
from jax.experimental.pallas import tpu as pltpu
from jax.experimental.pallas import tpu_sc as plsc

B = 16384
NC, NS, L = 2, 16, 16
NW = NC * NS
RPW = B // NW
CHUNKS = RPW // L
NF = 15
NFP = 16
BLK = 2048
NB = B // BLK

O_ER, O_EH, O_HS, O_FS, O_HP, O_CO = 0, 6, 594, 614, 622, 654
TAB = 688


_mesh = plsc.VectorSubcoreMesh(core_axis_name="c", subcore_axis_name="s",
                               num_cores=NC, num_subcores=NS)


@functools.partial(
    pl.kernel,
    out_type=jax.ShapeDtypeStruct((NB * NFP * BLK,), jnp.float32),
    mesh=_mesh,
    compiler_params=pltpu.CompilerParams(needs_layout_passes=False),
    scratch_types=[
        pltpu.VMEM((RPW,), jnp.int32),
        pltpu.VMEM((RPW,), jnp.int32),
        pltpu.VMEM((RPW,), jnp.int32),
        pltpu.VMEM((RPW,), jnp.int32),
        pltpu.VMEM((RPW,), jnp.int32),
        pltpu.VMEM((RPW,), jnp.int32),
        pltpu.VMEM((RPW,), jnp.int32),
        pltpu.VMEM((RPW,), jnp.int32),
        pltpu.VMEM((TAB,), jnp.float32),
        pltpu.VMEM((NFP * RPW,), jnp.float32),
        pltpu.SemaphoreType.DMA,
    ],
)
def _sc_gather(rp_h, cp_h, hand_h, hs_h, fs_h, hp_h, fp_h, co_h, tab_h, out_h,
               rp_b, cp_b, hand_b, hs_b, fs_b, hp_b, fp_b, co_b,
               tab_v, out_v, sem):
    wid = lax.axis_index("s") * NC + lax.axis_index("c")
    base = wid * RPW

    copies = [
        pltpu.async_copy(src.at[pl.ds(base, RPW)], dst, sem)
        for src, dst in ((rp_h, rp_b), (cp_h, cp_b), (hand_h, hand_b),
                         (hs_h, hs_b), (fs_h, fs_b), (hp_h, hp_b),
                         (fp_h, fp_b), (co_h, co_b))
    ]
    copies.append(pltpu.async_copy(tab_h, tab_v, sem))
    for c in copies:
        c.wait()

    def chunk(c, carry):
        s = c * L
        rp_v = rp_b[pl.ds(s, L)]
        cp_v = cp_b[pl.ds(s, L)]
        hand_v = hand_b[pl.ds(s, L)]
        hs_v = hs_b[pl.ds(s, L)]
        fs_v = fs_b[pl.ds(s, L)]
        hp_v = hp_b[pl.ds(s, L)]
        fp_v = fp_b[pl.ds(s, L)]
        co_v = co_b[pl.ds(s, L)]

        feats = []
        feats.append(plsc.load_gather(tab_v, [rp_v]))
        feats.append(plsc.load_gather(tab_v, [cp_v]))
        idx = hand_v * 3 + O_EH
        for _ in range(3):
            feats.append(plsc.load_gather(tab_v, [idx]))
            idx = idx + 1
        for src, width, off in ((hs_v, 2, O_HS), (fs_v, 2, O_FS),
                                (hp_v, 2, O_HP), (fp_v, 2, O_HP),
                                (co_v, 2, O_CO)):
            idx = src * width + off
            for _ in range(width):
                feats.append(plsc.load_gather(tab_v, [idx]))
                idx = idx + 1
        for k in range(NF):
            out_v[pl.ds(k * RPW + s, L)] = feats[k]
        return carry

    lax.fori_loop(0, CHUNKS, chunk, 0)

    j = wid // 4
    col = (wid % 4) * RPW
    out_copies = [
        pltpu.async_copy(out_v.at[pl.ds(k * RPW, RPW)],
                         out_h.at[pl.ds((j * NFP + k) * BLK + col, RPW)], sem)
        for k in range(NF)
    ]
    for c in out_copies:
        c.wait()


def _post_body(f_ref, ct_ref, w1_ref, b1_ref, w2_ref, b2_ref, out_ref):
    ct_t = ct_ref[...]
    w1 = w1_ref[...]
    b1 = b1_ref[...]
    w2 = w2_ref[...]
    b2 = b2_ref[...]
    fk = [f_ref[k * 16:(k + 1) * 16, :] for k in range(NF)]
    rs = []
    for d in range(9):
        h = ct_t * w1[0, d] + b1[0, d]
        for k in range(NF):
            h = h + fk[k] * w1[1 + k, d]
        rs.append(jnp.maximum(h, 0.0))
    outs = []
    for e in range(3):
        o = rs[0] * w2[0, e] + b2[0, e]
        for d in range(1, 9):
            o = o + rs[d] * w2[d, e]
        outs.append(o)
    out_ref[...] = jnp.concatenate(outs, axis=0)


_post = pl.pallas_call(
    _post_body,
    grid=(NB,),
    in_specs=[
        pl.BlockSpec((NFP * 16, 128), lambda j: (j, 0)),
        pl.BlockSpec((16, 128), lambda j: (j, 0)),
        pl.BlockSpec((16, 9), lambda j: (0, 0)),
        pl.BlockSpec((1, 9), lambda j: (0, 0)),
        pl.BlockSpec((9, 3), lambda j: (0, 0)),
        pl.BlockSpec((1, 3), lambda j: (0, 0)),
    ],
    out_specs=pl.BlockSpec((3 * 16, 128), lambda j: (j, 0)),
    out_shape=jax.ShapeDtypeStruct((NB * 3 * 16, 128), jnp.float32),
)


def kernel(raiser_pos, caller_pos, checked_to, hand, hands_strength,
           flops_strength, hand_pot, flop_pot, cards_ord,
           emb_raiser, emb_hand, emb_hs, emb_fs, emb_hp, emb_ord,
           W1, b1, W2, b2):
    f32 = jnp.float32
    i32 = jnp.int32

    tab = jnp.concatenate([
        emb_raiser.astype(f32).reshape(6),
        emb_hand.astype(f32).reshape(588),
        emb_hs.astype(f32).reshape(20),
        emb_fs.astype(f32).reshape(8),
        emb_hp.astype(f32).reshape(32),
        emb_ord.astype(f32).reshape(20),
        jnp.zeros((TAB - O_CO - 20,), f32),
    ])

    feats = _sc_gather(raiser_pos.astype(i32), caller_pos.astype(i32),
                       hand.astype(i32), hands_strength.astype(i32),
                       flops_strength.astype(i32), hand_pot.astype(i32),
                       flop_pot.astype(i32), cards_ord.astype(i32), tab)

    out_t = _post(feats.reshape(NB * NFP * 16, 128),
                  checked_to.astype(f32).reshape(NB * 16, 128),
                  W1.astype(f32), b1.astype(f32).reshape(1, 9),
                  W2.astype(f32), b2.astype(f32).reshape(1, 3))

    return (out_t.reshape(NB, 3, 16, 128).transpose(0, 2, 3, 1)
            .reshape(B, 3))


# speedup vs baseline: 11.6712x; 1.1156x over previous
import functools

import jax
import jax.numpy as jnp
from jax import lax
from jax.experimental import pallas as pl
from jax.experimental.pallas import tpu as pltpu
from jax.experimental.pallas import tpu_sc as plsc

B = 16384
NC, NS, L = 2, 16, 16
NW = NC * NS
RPW = B // NW
CHUNKS = RPW // L
NF = 15
NFP = 16
BLK = 2048
NB = B // BLK

O_ER, O_EH, O_HS, O_FS, O_HP, O_CO = 0, 6, 594, 614, 622, 654
TAB = 688


_mesh = plsc.VectorSubcoreMesh(core_axis_name="c", subcore_axis_name="s",
                               num_cores=NC, num_subcores=NS)


@functools.partial(
    pl.kernel,
    out_type=jax.ShapeDtypeStruct((NB * NFP * BLK,), jnp.float32),
    mesh=_mesh,
    compiler_params=pltpu.CompilerParams(needs_layout_passes=False),
    scratch_types=[
        pltpu.VMEM((RPW,), jnp.int32),
        pltpu.VMEM((RPW,), jnp.int32),
        pltpu.VMEM((RPW,), jnp.int32),
        pltpu.VMEM((RPW,), jnp.int32),
        pltpu.VMEM((RPW,), jnp.int32),
        pltpu.VMEM((RPW,), jnp.int32),
        pltpu.VMEM((RPW,), jnp.int32),
        pltpu.VMEM((RPW,), jnp.int32),
        pltpu.VMEM((TAB,), jnp.float32),
        pltpu.VMEM((NFP * RPW,), jnp.float32),
        pltpu.SemaphoreType.DMA,
    ],
)
def _sc_gather(rp_h, cp_h, hand_h, hs_h, fs_h, hp_h, fp_h, co_h, tab_h, out_h,
               rp_b, cp_b, hand_b, hs_b, fs_b, hp_b, fp_b, co_b,
               tab_v, out_v, sem):
    wid = lax.axis_index("s") * NC + lax.axis_index("c")
    base = wid * RPW

    copies = [
        pltpu.async_copy(src.at[pl.ds(base, RPW)], dst, sem)
        for src, dst in ((rp_h, rp_b), (cp_h, cp_b), (hand_h, hand_b),
                         (hs_h, hs_b), (fs_h, fs_b), (hp_h, hp_b),
                         (fp_h, fp_b), (co_h, co_b))
    ]
    copies.append(pltpu.async_copy(tab_h, tab_v, sem))
    for c in copies:
        c.wait()

    def chunk(c, carry):
        s = c * L
        rp_v = rp_b[pl.ds(s, L)]
        cp_v = cp_b[pl.ds(s, L)]
        hand_v = hand_b[pl.ds(s, L)]
        hs_v = hs_b[pl.ds(s, L)]
        fs_v = fs_b[pl.ds(s, L)]
        hp_v = hp_b[pl.ds(s, L)]
        fp_v = fp_b[pl.ds(s, L)]
        co_v = co_b[pl.ds(s, L)]

        feats = []
        feats.append(plsc.load_gather(tab_v, [rp_v]))
        feats.append(plsc.load_gather(tab_v, [cp_v]))
        idx = hand_v * 3 + O_EH
        for _ in range(3):
            feats.append(plsc.load_gather(tab_v, [idx]))
            idx = idx + 1
        for src, width, off in ((hs_v, 2, O_HS), (fs_v, 2, O_FS),
                                (hp_v, 2, O_HP), (fp_v, 2, O_HP),
                                (co_v, 2, O_CO)):
            idx = src * width + off
            for _ in range(width):
                feats.append(plsc.load_gather(tab_v, [idx]))
                idx = idx + 1
        for k in range(NF):
            out_v[pl.ds(k * RPW + s, L)] = feats[k]
        return carry

    lax.fori_loop(0, CHUNKS, chunk, 0)

    j = wid // 4
    col = (wid % 4) * RPW
    out_copies = [
        pltpu.async_copy(out_v.at[pl.ds(k * RPW, RPW)],
                         out_h.at[pl.ds((j * NFP + k) * BLK + col, RPW)], sem)
        for k in range(NF)
    ]
    for c in out_copies:
        c.wait()


def _post_body(f_ref, ct_ref, w1_ref, b1_ref, w2_ref, b2_ref, out_ref):
    w1 = w1_ref[...]
    b1 = b1_ref[...]
    w2 = w2_ref[...]
    b2 = b2_ref[...]
    for j in range(NB):
        ct_t = ct_ref[j * 16:(j + 1) * 16, :]
        fk = [f_ref[(j * NFP + k) * 16:(j * NFP + k + 1) * 16, :]
              for k in range(NF)]
        rs = []
        for d in range(9):
            h = ct_t * w1[0, d] + b1[0, d]
            for k in range(NF):
                h = h + fk[k] * w1[1 + k, d]
            rs.append(jnp.maximum(h, 0.0))
        for e in range(3):
            o = rs[0] * w2[0, e] + b2[0, e]
            for d in range(1, 9):
                o = o + rs[d] * w2[d, e]
            out_ref[(j * 3 + e) * 16:(j * 3 + e + 1) * 16, :] = o


_post = pl.pallas_call(
    _post_body,
    out_shape=jax.ShapeDtypeStruct((NB * 3 * 16, 128), jnp.float32),
)


def kernel(raiser_pos, caller_pos, checked_to, hand, hands_strength,
           flops_strength, hand_pot, flop_pot, cards_ord,
           emb_raiser, emb_hand, emb_hs, emb_fs, emb_hp, emb_ord,
           W1, b1, W2, b2):
    f32 = jnp.float32
    i32 = jnp.int32

    tab = jnp.concatenate([
        emb_raiser.astype(f32).reshape(6),
        emb_hand.astype(f32).reshape(588),
        emb_hs.astype(f32).reshape(20),
        emb_fs.astype(f32).reshape(8),
        emb_hp.astype(f32).reshape(32),
        emb_ord.astype(f32).reshape(20),
        jnp.zeros((TAB - O_CO - 20,), f32),
    ])

    feats = _sc_gather(raiser_pos.astype(i32), caller_pos.astype(i32),
                       hand.astype(i32), hands_strength.astype(i32),
                       flops_strength.astype(i32), hand_pot.astype(i32),
                       flop_pot.astype(i32), cards_ord.astype(i32), tab)

    out_t = _post(feats.reshape(NB * NFP * 16, 128),
                  checked_to.astype(f32).reshape(NB * 16, 128),
                  W1.astype(f32), b1.astype(f32).reshape(1, 9),
                  W2.astype(f32), b2.astype(f32).reshape(1, 3))

    return (out_t.reshape(NB, 3, 16, 128).transpose(0, 2, 3, 1)
            .reshape(B, 3))


# speedup vs baseline: 11.7752x; 1.0089x over previous
import functools

import jax
import jax.numpy as jnp
from jax import lax
from jax.experimental import pallas as pl
from jax.experimental.pallas import tpu as pltpu
from jax.experimental.pallas import tpu_sc as plsc

B = 16384
NC, NS, L = 2, 16, 16
NW = NC * NS
RPW = B // NW
CHUNKS = RPW // L
NF = 15
NFP = 16
BLK = 2048
NB = B // BLK

O_ER, O_EH, O_HS, O_FS, O_HP, O_CO = 0, 6, 594, 614, 622, 654
TAB = 688


_mesh = plsc.VectorSubcoreMesh(core_axis_name="c", subcore_axis_name="s",
                               num_cores=NC, num_subcores=NS)


@functools.partial(
    pl.kernel,
    out_type=jax.ShapeDtypeStruct((NB * NFP * BLK,), jnp.float32),
    mesh=_mesh,
    compiler_params=pltpu.CompilerParams(needs_layout_passes=False),
    scratch_types=[
        pltpu.VMEM((RPW,), jnp.int32),
        pltpu.VMEM((RPW,), jnp.int32),
        pltpu.VMEM((RPW,), jnp.int32),
        pltpu.VMEM((RPW,), jnp.int32),
        pltpu.VMEM((RPW,), jnp.int32),
        pltpu.VMEM((RPW,), jnp.int32),
        pltpu.VMEM((RPW,), jnp.int32),
        pltpu.VMEM((RPW,), jnp.int32),
        pltpu.VMEM((TAB,), jnp.float32),
        pltpu.VMEM((NFP * RPW,), jnp.float32),
        pltpu.SemaphoreType.DMA,
    ],
)
def _sc_gather(rp_h, cp_h, hand_h, hs_h, fs_h, hp_h, fp_h, co_h, tab_h, out_h,
               rp_b, cp_b, hand_b, hs_b, fs_b, hp_b, fp_b, co_b,
               tab_v, out_v, sem):
    wid = lax.axis_index("s") * NC + lax.axis_index("c")
    base = wid * RPW

    copies = [
        pltpu.async_copy(src.at[pl.ds(base, RPW)], dst, sem)
        for src, dst in ((rp_h, rp_b), (cp_h, cp_b), (hand_h, hand_b),
                         (hs_h, hs_b), (fs_h, fs_b), (hp_h, hp_b),
                         (fp_h, fp_b), (co_h, co_b))
    ]
    copies.append(pltpu.async_copy(tab_h, tab_v, sem))
    for c in copies:
        c.wait()

    @plsc.parallel_loop(0, CHUNKS, 1, unroll=2)
    def chunk(c):
        s = c * L
        rp_v = rp_b[pl.ds(s, L)]
        cp_v = cp_b[pl.ds(s, L)]
        hand_v = hand_b[pl.ds(s, L)]
        hs_v = hs_b[pl.ds(s, L)]
        fs_v = fs_b[pl.ds(s, L)]
        hp_v = hp_b[pl.ds(s, L)]
        fp_v = fp_b[pl.ds(s, L)]
        co_v = co_b[pl.ds(s, L)]

        feats = []
        feats.append(plsc.load_gather(tab_v, [rp_v]))
        feats.append(plsc.load_gather(tab_v, [cp_v]))
        idx = hand_v * 3 + O_EH
        for _ in range(3):
            feats.append(plsc.load_gather(tab_v, [idx]))
            idx = idx + 1
        for src, width, off in ((hs_v, 2, O_HS), (fs_v, 2, O_FS),
                                (hp_v, 2, O_HP), (fp_v, 2, O_HP),
                                (co_v, 2, O_CO)):
            idx = src * width + off
            for _ in range(width):
                feats.append(plsc.load_gather(tab_v, [idx]))
                idx = idx + 1
        for k in range(NF):
            out_v[pl.ds(k * RPW + s, L)] = feats[k]

    j = wid // 4
    col = (wid % 4) * RPW
    out_copies = [
        pltpu.async_copy(out_v.at[pl.ds(k * RPW, RPW)],
                         out_h.at[pl.ds((j * NFP + k) * BLK + col, RPW)], sem)
        for k in range(NF)
    ]
    for c in out_copies:
        c.wait()


def _post_body(f_ref, ct_ref, w1_ref, b1_ref, w2_ref, b2_ref, out_ref):
    w1 = w1_ref[...]
    b1 = b1_ref[...]
    w2 = w2_ref[...]
    b2 = b2_ref[...]
    for j in range(NB):
        ct_t = ct_ref[j * 16:(j + 1) * 16, :]
        fk = [f_ref[(j * NFP + k) * 16:(j * NFP + k + 1) * 16, :]
              for k in range(NF)]
        rs = []
        for d in range(9):
            h = ct_t * w1[0, d] + b1[0, d]
            for k in range(NF):
                h = h + fk[k] * w1[1 + k, d]
            rs.append(jnp.maximum(h, 0.0))
        for e in range(3):
            o = rs[0] * w2[0, e] + b2[0, e]
            for d in range(1, 9):
                o = o + rs[d] * w2[d, e]
            out_ref[(j * 3 + e) * 16:(j * 3 + e + 1) * 16, :] = o


_post = pl.pallas_call(
    _post_body,
    out_shape=jax.ShapeDtypeStruct((NB * 3 * 16, 128), jnp.float32),
)


def kernel(raiser_pos, caller_pos, checked_to, hand, hands_strength,
           flops_strength, hand_pot, flop_pot, cards_ord,
           emb_raiser, emb_hand, emb_hs, emb_fs, emb_hp, emb_ord,
           W1, b1, W2, b2):
    f32 = jnp.float32
    i32 = jnp.int32

    def _placed(t, off, n):
        return jnp.pad(t.astype(f32).reshape(n), (off, TAB - off - n))

    tab = (_placed(emb_raiser, O_ER, 6) + _placed(emb_hand, O_EH, 588)
           + _placed(emb_hs, O_HS, 20) + _placed(emb_fs, O_FS, 8)
           + _placed(emb_hp, O_HP, 32) + _placed(emb_ord, O_CO, 20))

    feats = _sc_gather(raiser_pos.astype(i32), caller_pos.astype(i32),
                       hand.astype(i32), hands_strength.astype(i32),
                       flops_strength.astype(i32), hand_pot.astype(i32),
                       flop_pot.astype(i32), cards_ord.astype(i32), tab)

    out_t = _post(feats.reshape(NB * NFP * 16, 128),
                  checked_to.astype(f32).reshape(NB * 16, 128),
                  W1.astype(f32), b1.astype(f32).reshape(1, 9),
                  W2.astype(f32), b2.astype(f32).reshape(1, 3))

    return (out_t.reshape(NB, 3, 16, 128).transpose(0, 2, 3, 1)
            .reshape(B, 3))


# speedup vs baseline: 12.2288x; 1.0385x over previous
import functools

import jax
import jax.numpy as jnp
from jax import lax
from jax.experimental import pallas as pl
from jax.experimental.pallas import tpu as pltpu
from jax.experimental.pallas import tpu_sc as plsc

B = 16384
NC, NS, L = 2, 16, 16
NW = NC * NS
RPW = B // NW
CHUNKS = RPW // L
NF = 15
NFP = 16
BLK = 2048
NB = B // BLK

O_ER, O_EH, O_HS, O_FS, O_HP, O_CO = 0, 6, 594, 614, 622, 654
TAB = 688


_mesh = plsc.VectorSubcoreMesh(core_axis_name="c", subcore_axis_name="s",
                               num_cores=NC, num_subcores=NS)


@functools.partial(
    pl.kernel,
    out_type=jax.ShapeDtypeStruct((NB * NFP * BLK,), jnp.float32),
    mesh=_mesh,
    compiler_params=pltpu.CompilerParams(needs_layout_passes=False),
    scratch_types=[
        pltpu.VMEM((RPW,), jnp.int32),
        pltpu.VMEM((RPW,), jnp.int32),
        pltpu.VMEM((RPW,), jnp.int32),
        pltpu.VMEM((RPW,), jnp.int32),
        pltpu.VMEM((RPW,), jnp.int32),
        pltpu.VMEM((RPW,), jnp.int32),
        pltpu.VMEM((RPW,), jnp.int32),
        pltpu.VMEM((RPW,), jnp.int32),
        pltpu.VMEM((TAB,), jnp.float32),
        pltpu.VMEM((NFP * RPW,), jnp.float32),
        pltpu.SemaphoreType.DMA,
    ],
)
def _sc_gather(rp_h, cp_h, hand_h, hs_h, fs_h, hp_h, fp_h, co_h, tab_h, out_h,
               rp_b, cp_b, hand_b, hs_b, fs_b, hp_b, fp_b, co_b,
               tab_v, out_v, sem):
    wid = lax.axis_index("s") * NC + lax.axis_index("c")
    base = wid * RPW

    copies = [
        pltpu.async_copy(src.at[pl.ds(base, RPW)], dst, sem)
        for src, dst in ((rp_h, rp_b), (cp_h, cp_b), (hand_h, hand_b),
                         (hs_h, hs_b), (fs_h, fs_b), (hp_h, hp_b),
                         (fp_h, fp_b), (co_h, co_b))
    ]
    copies.append(pltpu.async_copy(tab_h, tab_v, sem))
    for c in copies:
        c.wait()

    @plsc.parallel_loop(0, CHUNKS, 1, unroll=2)
    def chunk(c):
        s = c * L
        rp_v = rp_b[pl.ds(s, L)]
        cp_v = cp_b[pl.ds(s, L)]
        hand_v = hand_b[pl.ds(s, L)]
        hs_v = hs_b[pl.ds(s, L)]
        fs_v = fs_b[pl.ds(s, L)]
        hp_v = hp_b[pl.ds(s, L)]
        fp_v = fp_b[pl.ds(s, L)]
        co_v = co_b[pl.ds(s, L)]

        feats = []
        feats.append(plsc.load_gather(tab_v, [rp_v]))
        feats.append(plsc.load_gather(tab_v, [cp_v]))
        for src, width, vlen, off in (
                (hand_v, 3, 196, O_EH), (hs_v, 2, 10, O_HS),
                (fs_v, 2, 4, O_FS), (hp_v, 2, 16, O_HP),
                (fp_v, 2, 16, O_HP), (co_v, 2, 10, O_CO)):
            idx = src + off
            for w in range(width):
                feats.append(plsc.load_gather(tab_v, [idx]))
                if w < width - 1:
                    idx = idx + vlen
        for k in range(NF):
            out_v[pl.ds(k * RPW + s, L)] = feats[k]

    j = wid // 4
    col = (wid % 4) * RPW
    out_copies = [
        pltpu.async_copy(out_v.at[pl.ds(k * RPW, RPW)],
                         out_h.at[pl.ds((j * NFP + k) * BLK + col, RPW)], sem)
        for k in range(NF)
    ]
    for c in out_copies:
        c.wait()


def _post_body(f_ref, ct_ref, w1t_ref, b1_ref, w2t_ref, b2_ref, out_ref):
    w1t = w1t_ref[...]
    b1 = b1_ref[...]
    w2t = w2t_ref[...]
    b2 = b2_ref[...]
    for j in range(NB):
        ct_t = ct_ref[j * 16:(j + 1) * 16, :]
        fk = [f_ref[(j * NFP + k) * 16:(j * NFP + k + 1) * 16, :]
              for k in range(NF)]
        rs = []
        for d in range(9):
            h = ct_t * w1t[d, 0] + b1[0, d]
            for k in range(NF):
                h = h + fk[k] * w1t[d, 1 + k]
            rs.append(jnp.maximum(h, 0.0))
        for e in range(3):
            o = rs[0] * w2t[e, 0] + b2[0, e]
            for d in range(1, 9):
                o = o + rs[d] * w2t[e, d]
            out_ref[(j * 3 + e) * 16:(j * 3 + e + 1) * 16, :] = o


_post = pl.pallas_call(
    _post_body,
    out_shape=jax.ShapeDtypeStruct((NB * 3 * 16, 128), jnp.float32),
)


def kernel(raiser_pos, caller_pos, checked_to, hand, hands_strength,
           flops_strength, hand_pot, flop_pot, cards_ord,
           emb_raiser, emb_hand, emb_hs, emb_fs, emb_hp, emb_ord,
           W1, b1, W2, b2):
    f32 = jnp.float32
    i32 = jnp.int32

    def _placed(t, off, n):
        return jnp.pad(t.T.astype(f32).reshape(n), (off, TAB - off - n))

    tab = (_placed(emb_raiser, O_ER, 6) + _placed(emb_hand, O_EH, 588)
           + _placed(emb_hs, O_HS, 20) + _placed(emb_fs, O_FS, 8)
           + _placed(emb_hp, O_HP, 32) + _placed(emb_ord, O_CO, 20))

    feats = _sc_gather(raiser_pos.astype(i32), caller_pos.astype(i32),
                       hand.astype(i32), hands_strength.astype(i32),
                       flops_strength.astype(i32), hand_pot.astype(i32),
                       flop_pot.astype(i32), cards_ord.astype(i32), tab)

    out_t = _post(feats.reshape(NB * NFP * 16, 128),
                  checked_to.astype(f32).reshape(NB * 16, 128),
                  W1.astype(f32).T, b1.astype(f32).reshape(1, 9),
                  W2.astype(f32).T, b2.astype(f32).reshape(1, 3))

    return (out_t.reshape(NB, 3, 16, 128).transpose(0, 2, 3, 1)
            .reshape(B, 3))
